# Initial kernel scaffold; baseline (speedup 1.0000x reference)
import jax
import jax.numpy as jnp
from jax.experimental import pallas as pl


def kernel(h_lig, h_rec, x_lig, x_rec, edge_index_ll, edge_index_rl, W1e_ll, b1e_ll, W2e_ll, b2e_ll, W1c_ll, b1c_ll, W2c_ll, b2c_ll, W1e_rl, b1e_rl, W2e_rl, b2e_rl, W1c_rl, b1c_rl, W2c_rl, b2c_rl, Wn1, bn1, Wn2, bn2):
    raise NotImplementedError("write your pallas kernel here")


# speedup vs baseline: 2.2702x; 2.2702x over previous
import functools

import jax
import jax.numpy as jnp
from jax import lax
from jax.experimental import pallas as pl
from jax.experimental.pallas import tpu as pltpu
from jax.experimental.pallas import tpu_sc as plsc

_PREC = lax.Precision.HIGHEST
_NW = 32
_CHUNK = 128


def _silu(x):
    return x / (1.0 + jnp.exp(-x))


def _row_block(n, cap):
    b = 8
    for c in range(8, cap + 1, 8):
        if n % c == 0:
            b = c
    return b


def _rowmm(x, w, b, xpad, splits):
    n, d = x.shape
    m = w.shape[1]
    dx = xpad.shape[1]
    bn = _row_block(n, 2048)
    sw = m // splits

    def body(x_ref, w_ref, b_ref, xp_ref, *outs):
        acc = jnp.dot(x_ref[...], w_ref[...], precision=_PREC,
                      preferred_element_type=jnp.float32) + b_ref[...]
        xp = xp_ref[...]
        for j, o in enumerate(outs):
            o[...] = jnp.concatenate([acc[:, j * sw:(j + 1) * sw], xp], axis=1)

    outs = tuple(jax.ShapeDtypeStruct((n, sw + dx), jnp.float32)
                 for _ in range(splits))
    return pl.pallas_call(
        body,
        grid=(n // bn,),
        in_specs=[
            pl.BlockSpec((bn, d), lambda i: (i, 0)),
            pl.BlockSpec((d, m), lambda i: (0, 0)),
            pl.BlockSpec((1, m), lambda i: (0, 0)),
            pl.BlockSpec((bn, dx), lambda i: (i, 0)),
        ],
        out_specs=tuple(pl.BlockSpec((bn, sw + dx), lambda i: (i, 0))
                        for _ in range(splits)),
        out_shape=outs,
    )(x, w, b.reshape(1, m), xpad)


def _sc_gather(tsrc, tdst, src_idx, dst_idx):
    e = src_idx.shape[0]
    w = tsrc.shape[1]
    e_per_w = e // _NW
    n_chunks = e_per_w // _CHUNK
    mesh = plsc.VectorSubcoreMesh(core_axis_name="c", subcore_axis_name="s")

    @functools.partial(
        pl.kernel,
        out_type=(jax.ShapeDtypeStruct((e, w), jnp.float32),
                  jax.ShapeDtypeStruct((e, w), jnp.float32)),
        mesh=mesh,
        scratch_types=[
            pltpu.VMEM((_CHUNK,), jnp.int32),
            pltpu.VMEM((_CHUNK,), jnp.int32),
            pltpu.VMEM((_CHUNK, w), jnp.float32),
            pltpu.VMEM((_CHUNK, w), jnp.float32),
            pltpu.SemaphoreType.DMA,
            pltpu.SemaphoreType.DMA,
        ],
    )
    def k(tsrc_h, tdst_h, src_h, dst_h, gs_h, gd_h,
          idx_s, idx_d, rs, rd, sem_g, sem_w):
        wid = lax.axis_index("c") * 16 + lax.axis_index("s")
        base_w = wid * e_per_w

        def body(i, carry):
            base = base_w + i * _CHUNK
            pltpu.sync_copy(src_h.at[pl.ds(base, _CHUNK)], idx_s)
            pltpu.sync_copy(dst_h.at[pl.ds(base, _CHUNK)], idx_d)
            c1 = pltpu.async_copy(tsrc_h.at[idx_s], rs, sem_g)
            c2 = pltpu.async_copy(tdst_h.at[idx_d], rd, sem_g)
            c1.wait(); c2.wait()
            w1 = pltpu.async_copy(rs, gs_h.at[pl.ds(base, _CHUNK)], sem_w)
            w2 = pltpu.async_copy(rd, gd_h.at[pl.ds(base, _CHUNK)], sem_w)
            w1.wait(); w2.wait()
            return carry

        lax.fori_loop(0, n_chunks, body, 0)

    return k(tsrc, tdst, src_idx, dst_idx)


def _tc_edge(gs, gd, w2e, b2e, w2c, b2c, wde, wdc):
    e, w = gs.shape
    h = 128
    be = 1024

    def body(gs_ref, gd_ref, w2e_ref, b2e_ref, w2c_ref,
             b2c_ref, wde_ref, wdc_ref, oh_ref, ox_ref):
        gsv = gs_ref[...]
        gdv = gd_ref[...]
        xdiff = gsv[:, 2 * h:] - gdv[:, 2 * h:]
        d2 = jnp.sum(xdiff * xdiff, axis=1, keepdims=True)
        dij = jnp.sqrt(d2)
        xn = xdiff / (dij + 1e-9)
        ue = gsv[:, :h] + gdv[:, :h] + dij * wde_ref[...]
        uc = gsv[:, h:2 * h] + gdv[:, h:2 * h] + dij * wdc_ref[...]
        a = _silu(ue)
        mh = _silu(jnp.dot(a, w2e_ref[...], precision=_PREC,
                           preferred_element_type=jnp.float32) + b2e_ref[...])
        c = _silu(uc)
        s = _silu(jnp.sum(c * w2c_ref[...], axis=1, keepdims=True) + b2c_ref[...])
        oh_ref[...] = mh
        ox_ref[...] = s * xn

    return pl.pallas_call(
        body,
        grid=(e // be,),
        in_specs=[
            pl.BlockSpec((be, w), lambda i: (i, 0)),
            pl.BlockSpec((be, w), lambda i: (i, 0)),
            pl.BlockSpec((h, h), lambda i: (0, 0)),
            pl.BlockSpec((1, h), lambda i: (0, 0)),
            pl.BlockSpec((1, h), lambda i: (0, 0)),
            pl.BlockSpec((1, 1), lambda i: (0, 0)),
            pl.BlockSpec((1, h), lambda i: (0, 0)),
            pl.BlockSpec((1, h), lambda i: (0, 0)),
        ],
        out_specs=(pl.BlockSpec((be, h), lambda i: (i, 0)),
                   pl.BlockSpec((be, h), lambda i: (i, 0))),
        out_shape=(jax.ShapeDtypeStruct((e, h), jnp.float32),
                   jax.ShapeDtypeStruct((e, h), jnp.float32)),
    )(gs, gd, w2e, b2e.reshape(1, h), w2c.reshape(1, h),
      b2c.reshape(1, 1), wde.reshape(1, h), wdc.reshape(1, h))


def _sc_scatter(dst_ll, m_ll, dst_rl, m_rl, zeros_hbm):
    nacc, hh = zeros_hbm.shape
    e_ll = dst_ll.shape[0]
    e_rl = dst_rl.shape[0]
    rpt = nacc // 16
    mesh = plsc.VectorSubcoreMesh(core_axis_name="c", subcore_axis_name="s")

    @functools.partial(
        pl.kernel,
        out_type=jax.ShapeDtypeStruct((2, nacc, hh), jnp.float32),
        mesh=mesh,
        scratch_types=[
            pltpu.VMEM((_CHUNK,), jnp.int32),
            pltpu.VMEM((_CHUNK, hh), jnp.float32),
            pltpu.VMEM_SHARED((nacc, hh), jnp.float32),
        ],
    )
    def k(dll_h, mll_h, drl_h, mrl_h, z_h, o_h, idx_v, m_v, acc):
        cid = lax.axis_index("c")
        sid = lax.axis_index("s")
        wid = cid * 16 + sid
        r0 = sid * rpt
        pltpu.sync_copy(z_h.at[pl.ds(r0, rpt)], acc.at[pl.ds(r0, rpt)])
        plsc.subcore_barrier()

        def run(dst_h, m_h, e):
            e_per_w = e // _NW
            n_chunks = e_per_w // _CHUNK
            base_w = wid * e_per_w

            def body(i, carry):
                base = base_w + i * _CHUNK
                pltpu.sync_copy(dst_h.at[pl.ds(base, _CHUNK)], idx_v)
                pltpu.sync_copy(m_h.at[pl.ds(base, _CHUNK)], m_v)
                pltpu.sync_copy(m_v, acc.at[idx_v], add=True)
                return carry

            lax.fori_loop(0, n_chunks, body, 0)

        run(dll_h, mll_h, e_ll)
        run(drl_h, mrl_h, e_rl)
        plsc.subcore_barrier()
        pltpu.sync_copy(acc.at[pl.ds(r0, rpt)], o_h.at[cid, pl.ds(r0, rpt)])

    return k(dst_ll, m_ll, dst_rl, m_rl, zeros_hbm)


def _tc_final(h, xp, ah0, ah1, ax0, ax1, wn1a, wn1b, bn1, wn2, bn2):
    n, d = h.shape
    bn = _row_block(n, 2048)

    def body(h_ref, xp_ref, ah0_ref, ah1_ref, ax0_ref, ax1_ref,
             wn1a_ref, wn1b_ref, bn1_ref, wn2_ref, bn2_ref, oh_ref, ox_ref):
        hv = h_ref[...]
        hn = ah0_ref[...] + ah1_ref[...]
        t = _silu(jnp.dot(hv, wn1a_ref[...], precision=_PREC,
                          preferred_element_type=jnp.float32)
                  + jnp.dot(hn, wn1b_ref[...], precision=_PREC,
                            preferred_element_type=jnp.float32)
                  + bn1_ref[...])
        oh_ref[...] = hv + jnp.dot(t, wn2_ref[...], precision=_PREC,
                                   preferred_element_type=jnp.float32) + bn2_ref[...]
        ox_ref[...] = xp_ref[...] + ax0_ref[...] + ax1_ref[...]

    return pl.pallas_call(
        body,
        grid=(n // bn,),
        in_specs=[
            pl.BlockSpec((bn, d), lambda i: (i, 0)),
            pl.BlockSpec((bn, d), lambda i: (i, 0)),
            pl.BlockSpec((bn, d), lambda i: (i, 0)),
            pl.BlockSpec((bn, d), lambda i: (i, 0)),
            pl.BlockSpec((bn, d), lambda i: (i, 0)),
            pl.BlockSpec((bn, d), lambda i: (i, 0)),
            pl.BlockSpec((d, d), lambda i: (0, 0)),
            pl.BlockSpec((d, d), lambda i: (0, 0)),
            pl.BlockSpec((1, d), lambda i: (0, 0)),
            pl.BlockSpec((d, d), lambda i: (0, 0)),
            pl.BlockSpec((1, d), lambda i: (0, 0)),
        ],
        out_specs=(pl.BlockSpec((bn, d), lambda i: (i, 0)),
                   pl.BlockSpec((bn, d), lambda i: (i, 0))),
        out_shape=(jax.ShapeDtypeStruct((n, d), jnp.float32),
                   jax.ShapeDtypeStruct((n, d), jnp.float32)),
    )(h, xp, ah0, ah1, ax0, ax1, wn1a, wn1b, bn1.reshape(1, d), wn2,
      bn2.reshape(1, d))


def _pad_edges(src, dst, dummy):
    e = src.shape[0]
    gran = _NW * _CHUNK
    e_pad = -(-e // gran) * gran
    pad = e_pad - e
    if pad:
        src = jnp.concatenate([src, jnp.zeros((pad,), jnp.int32)])
        dst = jnp.concatenate([dst, jnp.full((pad,), dummy, jnp.int32)])
    return src, dst


def kernel(h_lig, h_rec, x_lig, x_rec, edge_index_ll, edge_index_rl,
           W1e_ll, b1e_ll, W2e_ll, b2e_ll, W1c_ll, b1c_ll, W2c_ll, b2c_ll,
           W1e_rl, b1e_rl, W2e_rl, b2e_rl, W1c_rl, b1c_rl, W2c_rl, b2c_rl,
           Wn1, bn1, Wn2, bn2):
    n_lig, d = h_lig.shape

    w_lig = jnp.concatenate(
        [W1e_ll[:d], W1c_ll[:d], W1e_ll[d:2 * d], W1c_ll[d:2 * d],
         W1e_rl[d:2 * d], W1c_rl[d:2 * d]], axis=1)
    b_lig = jnp.concatenate(
        [jnp.zeros((2 * d,), jnp.float32), b1e_ll, b1c_ll, b1e_rl, b1c_rl])
    x_lig_p = jnp.pad(x_lig, ((0, 0), (0, d - x_lig.shape[1])))
    x_rec_p = jnp.pad(x_rec, ((0, 0), (0, d - x_rec.shape[1])))
    t_src_ll, t_dst_ll, t_dst_rl = _rowmm(h_lig, w_lig, b_lig, x_lig_p, 3)
    w_rec = jnp.concatenate([W1e_rl[:d], W1c_rl[:d]], axis=1)
    (t_src_rl,) = _rowmm(h_rec, w_rec, jnp.zeros((2 * d,), jnp.float32),
                         x_rec_p, 1)

    src_ll, dst_ll = _pad_edges(edge_index_ll[0], edge_index_ll[1], n_lig)
    src_rl, dst_rl = _pad_edges(edge_index_rl[0], edge_index_rl[1], n_lig)
    gs_ll, gd_ll = _sc_gather(t_src_ll, t_dst_ll, src_ll, dst_ll)
    gs_rl, gd_rl = _sc_gather(t_src_rl, t_dst_rl, src_rl, dst_rl)

    mh_ll, mx_ll = _tc_edge(gs_ll, gd_ll, W2e_ll, b2e_ll, W2c_ll[:, 0], b2c_ll,
                            W1e_ll[2 * d], W1c_ll[2 * d])
    mh_rl, mx_rl = _tc_edge(gs_rl, gd_rl, W2e_rl, b2e_rl, W2c_rl[:, 0], b2c_rl,
                            W1e_rl[2 * d], W1c_rl[2 * d])

    nacc = -(-(n_lig + 1) // 128) * 128
    zeros_hbm = jnp.zeros((nacc, d), jnp.float32)
    acc_h = _sc_scatter(dst_ll, mh_ll, dst_rl, mh_rl, zeros_hbm)
    acc_x = _sc_scatter(dst_ll, mx_ll, dst_rl, mx_rl, zeros_hbm)

    h_out, xp_out = _tc_final(
        h_lig, x_lig_p,
        acc_h[0, :n_lig], acc_h[1, :n_lig], acc_x[0, :n_lig], acc_x[1, :n_lig],
        Wn1[:d], Wn1[d:], bn1, Wn2, bn2)

    return (h_out, h_rec, xp_out[:, :x_lig.shape[1]], x_rec)
